# baseline (speedup 1.0000x reference)
import functools

import jax
import jax.numpy as jnp
from jax.experimental import pallas as pl
from jax.experimental.pallas import tpu as pltpu

_NBUF = 3


def _gcn_body(x_hbm, w0_ref, b0_ref, w1_ref, b1_ref, adj_hbm,
              out_ref, yy_ref, h_ref, stats_ref, abuf_ref,
              sem_ref, xsem_ref, *, nb, block_m, n):
    i = pl.program_id(0)
    nsteps = 2 * nb

    def copy_for(step):
        blk = jax.lax.rem(step, nb)
        slot = jax.lax.rem(step, _NBUF)
        return pltpu.make_async_copy(
            adj_hbm.at[pl.ds(blk * block_m, block_m), :],
            abuf_ref.at[slot],
            sem_ref.at[slot],
        )

    @pl.when(i == 0)
    def _init():
        for s in range(_NBUF):
            copy_for(s).start()
        xcopy = pltpu.make_async_copy(x_hbm, h_ref, xsem_ref)
        xcopy.start()
        xcopy.wait()
        yy_ref[...] = jnp.dot(h_ref[...], w0_ref[...],
                              preferred_element_type=jnp.float32
                              ).astype(jnp.bfloat16)
        stats_ref[...] = jnp.zeros_like(stats_ref)

    copy_for(i).wait()
    slot = jax.lax.rem(i, _NBUF)

    def _do_layer1(adj_blk):
        h = jnp.dot(adj_blk.astype(jnp.bfloat16), yy_ref[...],
                    preferred_element_type=jnp.float32)
        h = jnp.maximum(h + b0_ref[...], 0.0)
        h_ref[pl.ds(i * block_m, block_m), :] = h
        stats_ref[0:1, :] += jnp.sum(h, axis=0, keepdims=True)
        stats_ref[1:2, :] += jnp.sum(h * h, axis=0, keepdims=True)

    for _s in range(_NBUF):
        @pl.when(jnp.logical_and(i < nb, slot == _s))
        def _l1(_s=_s):
            _do_layer1(abuf_ref[_s])

    @pl.when(i == nb)
    def _bn_project():
        mean = stats_ref[0:1, :] / n
        var = stats_ref[1:2, :] / n - mean * mean
        scale = jax.lax.rsqrt(var + 1e-5)
        hn = (h_ref[...] - mean) * scale
        yy_ref[...] = jnp.dot(hn.astype(jnp.bfloat16), w1_ref[...],
                              preferred_element_type=jnp.float32
                              ).astype(jnp.bfloat16)

    def _do_layer2(adj_blk):
        o = jnp.dot(adj_blk.astype(jnp.bfloat16), yy_ref[...],
                    preferred_element_type=jnp.float32)
        out_ref[...] = o + b1_ref[...]

    for _s in range(_NBUF):
        @pl.when(jnp.logical_and(i >= nb, slot == _s))
        def _l2(_s=_s):
            _do_layer2(abuf_ref[_s])

    @pl.when(i + _NBUF < nsteps)
    def _prefetch():
        copy_for(i + _NBUF).start()


@functools.partial(jax.jit, static_argnames=("block_m",))
def _gcn(x, adj, W0, b0, W1, b1, block_m=400):
    n, d = x.shape
    h_dim = W0.shape[1]
    o_dim = W1.shape[1]
    nb = n // block_m

    out = pl.pallas_call(
        functools.partial(_gcn_body, nb=nb, block_m=block_m, n=n),
        grid=(2 * nb,),
        in_specs=[
            pl.BlockSpec(memory_space=pltpu.HBM),
            pl.BlockSpec((d, h_dim), lambda i: (0, 0)),
            pl.BlockSpec((1, h_dim), lambda i: (0, 0)),
            pl.BlockSpec((h_dim, o_dim), lambda i: (0, 0)),
            pl.BlockSpec((1, o_dim), lambda i: (0, 0)),
            pl.BlockSpec(memory_space=pltpu.HBM),
        ],
        out_specs=pl.BlockSpec((block_m, o_dim),
                               lambda i: (jnp.maximum(i - nb, 0), 0)),
        out_shape=jax.ShapeDtypeStruct((n, o_dim), jnp.float32),
        scratch_shapes=[
            pltpu.VMEM((n, h_dim), jnp.bfloat16),
            pltpu.VMEM((n, h_dim), jnp.float32),
            pltpu.VMEM((8, h_dim), jnp.float32),
            pltpu.VMEM((_NBUF, block_m, n), jnp.float32),
            pltpu.SemaphoreType.DMA((_NBUF,)),
            pltpu.SemaphoreType.DMA,
        ],
    )(x, W0, b0.reshape(1, h_dim), W1, b1.reshape(1, o_dim), adj)
    return out


def kernel(x, adj, W0, b0, W1, b1):
    return _gcn(x, adj, W0, b0, W1, b1)


# speedup vs baseline: 1.0349x; 1.0349x over previous
import functools

import jax
import jax.numpy as jnp
from jax.experimental import pallas as pl
from jax.experimental.pallas import tpu as pltpu


def _gcn_body(x_ref, w0_ref, b0_ref, w1_ref, b1_ref, adj_ref,
              out_ref, y0_ref, h_ref, y1_ref, stats_ref, *, nb, block_m, n):
    i = pl.program_id(0)

    @pl.when(i == 0)
    def _init():
        y0_ref[...] = jnp.dot(x_ref[...], w0_ref[...],
                              preferred_element_type=jnp.float32
                              ).astype(jnp.bfloat16)
        stats_ref[...] = jnp.zeros_like(stats_ref)

    @pl.when(i < nb)
    def _layer1():
        h = jnp.dot(adj_ref[...].astype(jnp.bfloat16), y0_ref[...],
                    preferred_element_type=jnp.float32)
        h = jnp.maximum(h + b0_ref[...], 0.0)
        h_ref[pl.ds(i * block_m, block_m), :] = h
        stats_ref[0:1, :] += jnp.sum(h, axis=0, keepdims=True)
        stats_ref[1:2, :] += jnp.sum(h * h, axis=0, keepdims=True)

    @pl.when(i == nb)
    def _bn_project():
        mean = stats_ref[0:1, :] / n
        var = stats_ref[1:2, :] / n - mean * mean
        scale = jax.lax.rsqrt(var + 1e-5)
        hn = (h_ref[...] - mean) * scale
        y1_ref[...] = jnp.dot(hn.astype(jnp.bfloat16), w1_ref[...],
                              preferred_element_type=jnp.float32
                              ).astype(jnp.bfloat16)

    @pl.when(i >= nb)
    def _layer2():
        o = jnp.dot(adj_ref[...].astype(jnp.bfloat16), y1_ref[...],
                    preferred_element_type=jnp.float32)
        out_ref[...] = o + b1_ref[...]


@functools.partial(jax.jit, static_argnames=("block_m",))
def _gcn(x, adj, W0, b0, W1, b1, block_m=400):
    n, d = x.shape
    h_dim = W0.shape[1]
    o_dim = W1.shape[1]
    nb = n // block_m

    out = pl.pallas_call(
        functools.partial(_gcn_body, nb=nb, block_m=block_m, n=n),
        grid=(2 * nb,),
        in_specs=[
            pl.BlockSpec((n, d), lambda i: (0, 0)),
            pl.BlockSpec((d, h_dim), lambda i: (0, 0)),
            pl.BlockSpec((1, h_dim), lambda i: (0, 0)),
            pl.BlockSpec((h_dim, o_dim), lambda i: (0, 0)),
            pl.BlockSpec((1, o_dim), lambda i: (0, 0)),
            pl.BlockSpec((block_m, n),
                         lambda i: (jax.lax.rem(i, nb), 0)),
        ],
        out_specs=pl.BlockSpec((block_m, o_dim),
                               lambda i: (jnp.maximum(i - nb, 0), 0)),
        out_shape=jax.ShapeDtypeStruct((n, o_dim), jnp.float32),
        scratch_shapes=[
            pltpu.VMEM((n, h_dim), jnp.bfloat16),
            pltpu.VMEM((n, h_dim), jnp.float32),
            pltpu.VMEM((n, o_dim), jnp.bfloat16),
            pltpu.VMEM((8, h_dim), jnp.float32),
        ],
    )(x, W0, b0.reshape(1, h_dim), W1, b1.reshape(1, o_dim), adj)
    return out


def kernel(x, adj, W0, b0, W1, b1):
    return _gcn(x, adj, W0, b0, W1, b1)


# speedup vs baseline: 1.0452x; 1.0100x over previous
import functools

import jax
import jax.numpy as jnp
from jax.experimental import pallas as pl
from jax.experimental.pallas import tpu as pltpu


def _gcn_body(x_ref, w0_ref, b0_ref, w1_ref, b1_ref, adj_ref,
              out_ref, y0_ref, h_ref, y1_ref, stats_ref, *, nb, block_m, n):
    i = pl.program_id(0)

    @pl.when(i == 0)
    def _init():
        y0_ref[...] = jnp.dot(x_ref[...], w0_ref[...],
                              preferred_element_type=jnp.float32
                              ).astype(jnp.bfloat16)
        stats_ref[...] = jnp.zeros_like(stats_ref)

    @pl.when(i < nb)
    def _layer1():
        h = jnp.dot(adj_ref[...].astype(jnp.bfloat16), y0_ref[...],
                    preferred_element_type=jnp.float32)
        h = jnp.maximum(h + b0_ref[...], 0.0)
        h_ref[pl.ds(i * block_m, block_m), :] = h
        stats_ref[0:1, :] += jnp.sum(h, axis=0, keepdims=True)
        stats_ref[1:2, :] += jnp.sum(h * h, axis=0, keepdims=True)

    @pl.when(i == nb)
    def _bn_project():
        mean = stats_ref[0:1, :] / n
        var = stats_ref[1:2, :] / n - mean * mean
        scale = jax.lax.rsqrt(var + 1e-5)
        hn = (h_ref[...] - mean) * scale
        y1_ref[...] = jnp.dot(hn.astype(jnp.bfloat16), w1_ref[...],
                              preferred_element_type=jnp.float32
                              ).astype(jnp.bfloat16)

    @pl.when(i >= nb)
    def _layer2():
        o = jnp.dot(adj_ref[...].astype(jnp.bfloat16), y1_ref[...],
                    preferred_element_type=jnp.float32)
        out_ref[...] = o + b1_ref[...]


@functools.partial(jax.jit, static_argnames=("block_m",))
def _gcn(x, adj, W0, b0, W1, b1, block_m=400):
    n, d = x.shape
    h_dim = W0.shape[1]
    o_dim = W1.shape[1]
    nb = n // block_m

    out = pl.pallas_call(
        functools.partial(_gcn_body, nb=nb, block_m=block_m, n=n),
        grid=(2 * nb,),
        in_specs=[
            pl.BlockSpec((n, d), lambda i: (0, 0)),
            pl.BlockSpec((d, h_dim), lambda i: (0, 0)),
            pl.BlockSpec((1, h_dim), lambda i: (0, 0)),
            pl.BlockSpec((h_dim, o_dim), lambda i: (0, 0)),
            pl.BlockSpec((1, o_dim), lambda i: (0, 0)),
            pl.BlockSpec((block_m, n),
                         lambda i: (jnp.minimum(i, 2 * nb - 1 - i), 0)),
        ],
        out_specs=pl.BlockSpec((block_m, o_dim),
                               lambda i: (jnp.minimum(nb - 1, 2 * nb - 1 - i),
                                          0)),
        out_shape=jax.ShapeDtypeStruct((n, o_dim), jnp.float32),
        scratch_shapes=[
            pltpu.VMEM((n, h_dim), jnp.bfloat16),
            pltpu.VMEM((n, h_dim), jnp.float32),
            pltpu.VMEM((n, o_dim), jnp.bfloat16),
            pltpu.VMEM((8, h_dim), jnp.float32),
        ],
    )(x, W0, b0.reshape(1, h_dim), W1, b1.reshape(1, o_dim), adj)
    return out


def kernel(x, adj, W0, b0, W1, b1):
    return _gcn(x, adj, W0, b0, W1, b1)
